# baseline (speedup 1.0000x reference)
import dataclasses
import functools
import jax
import jax.numpy as jnp
from jax import lax
from jax.experimental import pallas as pl
from jax.experimental.pallas import tpu as pltpu
from jax.experimental.pallas import tpu_sc as plsc

ROW = 8
LANES = 16
W = 2048
SLAB = 256
NSLAB = W // SLAB
LOOK = 6
TCHUNK = 2096
TROUNDS = 3


def kernel(s, arclengths, curve_control_points):
    n_seg = curve_control_points.shape[0]
    b = s.shape[0]
    table_t = curve_control_points.reshape(n_seg, ROW).T
    mesh = plsc.VectorSubcoreMesh(core_axis_name="c", subcore_axis_name="s")
    cp = pltpu.CompilerParams()
    if "needs_layout_passes" in pltpu.CompilerParams.__dataclass_fields__:
        cp = dataclasses.replace(cp, needs_layout_passes=False)
    if "use_tc_tiling_on_sc" in pltpu.CompilerParams.__dataclass_fields__:
        cp = dataclasses.replace(cp, use_tc_tiling_on_sc=False)

    @functools.partial(
        pl.kernel,
        mesh=mesh,
        compiler_params=cp,
        out_type=(
            jax.ShapeDtypeStruct((b // 128, 2, 128), jnp.float32),
            jax.ShapeDtypeStruct((b,), jnp.int32),
            jax.ShapeDtypeStruct((n_seg, ROW), jnp.float32),
        ),
        scratch_types=[
            pltpu.VMEM((W,), jnp.int32),
            pltpu.VMEM((W, ROW), jnp.float32),
            pltpu.VMEM((2, ROW, TCHUNK), jnp.float32),
            pltpu.VMEM((2, TCHUNK, ROW), jnp.float32),
        ]
        + [pltpu.SemaphoreType.DMA] * (LOOK + 5),
    )
    def run(s_hbm, tt_hbm, pos_hbm, idx_hbm, tbl_hbm, idxs_v, rows_v, tin_v, tout_v, *sems):
        sid = lax.axis_index("s")
        in_sems = sems[LOOK + 1 : LOOK + 3]
        out_sems = sems[LOOK + 3 : LOOK + 5]

        def chunk_start(r):
            cid = r * 16 + sid
            c0 = jnp.minimum(cid * TCHUNK, n_seg - TCHUNK)
            return pl.multiple_of(c0, 8)

        def fetch(r):
            return pltpu.async_copy(
                tt_hbm.at[:, pl.ds(chunk_start(r), TCHUNK)],
                tin_v.at[r % 2],
                in_sems[r % 2],
            )

        h_in = [None] * TROUNDS
        h_out = [None] * TROUNDS
        h_in[0] = fetch(0)
        for r in range(TROUNDS):
            if r + 1 < TROUNDS:
                h_in[r + 1] = fetch(r + 1)
            h_in[r].wait()
            if r >= 2:
                h_out[r - 2].wait()

            @pl.loop(0, TCHUNK, step=LANES)
            def _(o):
                rid = o + lax.iota(jnp.int32, LANES)
                for j in range(ROW):
                    v = tin_v[r % 2, j, pl.ds(o, LANES)]
                    plsc.store_scatter(
                        tout_v.at[r % 2], [rid, jnp.full((LANES,), j, jnp.int32)], v
                    )

            h_out[r] = pltpu.async_copy(
                tout_v.at[r % 2],
                tbl_hbm.at[pl.ds(chunk_start(r), TCHUNK)],
                out_sems[r % 2],
            )
        for r in range(max(0, TROUNDS - 2), TROUNDS):
            h_out[r].wait()
        plsc.subcore_barrier()

        def body(s_blk, pos_blk, idx_blk):
            def pass_a(k):
                @plsc.parallel_loop(k * SLAB, (k + 1) * SLAB, step=LANES, unroll=2)
                def _(o):
                    sv = s_blk[pl.ds(o, LANES)]
                    ii = jnp.minimum(sv.astype(jnp.int32), n_seg - 1)
                    ii = jnp.maximum(ii, 0)
                    idxs_v[pl.ds(o, LANES)] = ii
                    idx_blk[pl.ds(o, LANES)] = ii

            def issue(k):
                return pltpu.async_copy(
                    tbl_hbm.at[idxs_v.at[pl.ds(k * SLAB, SLAB)]],
                    rows_v.at[pl.ds(k * SLAB, SLAB)],
                    sems[k % LOOK],
                )

            def pass_b(k):
                @plsc.parallel_loop(k * SLAB, (k + 1) * SLAB, step=LANES, unroll=2)
                def _(o):
                    sv = s_blk[pl.ds(o, LANES)]
                    fi = idxs_v[pl.ds(o, LANES)].astype(jnp.float32)
                    t = sv - fi
                    u = 1.0 - t
                    t2 = t * t
                    u2 = u * u
                    b0 = u2 * u
                    b1 = 3.0 * t * u2
                    b2 = 3.0 * t2 * u
                    b3 = t2 * t
                    rid = o + lax.iota(jnp.int32, LANES)
                    c = [
                        plsc.load_gather(
                            rows_v, [rid, jnp.full((LANES,), j, jnp.int32)]
                        )
                        for j in range(ROW)
                    ]
                    px = b0 * c[0] + b1 * c[2] + b2 * c[4] + b3 * c[6]
                    py = b0 * c[1] + b1 * c[3] + b2 * c[5] + b3 * c[7]
                    blk = o // 128
                    off = o - blk * 128
                    pos_blk[blk, 0, pl.ds(off, LANES)] = px
                    pos_blk[blk, 1, pl.ds(off, LANES)] = py

            handles = [None] * NSLAB
            for k in range(NSLAB):
                pass_a(k)
                if k >= LOOK:
                    handles[k - LOOK].wait()
                handles[k] = issue(k)
                if k >= LOOK:
                    pass_b(k - LOOK)
            for k in range(NSLAB - LOOK, NSLAB):
                handles[k].wait()
                pass_b(k)

        pltpu.emit_pipeline(
            body,
            grid=(b // W,),
            in_specs=[pl.BlockSpec((W,), lambda i: (i,))],
            out_specs=[
                pl.BlockSpec((W // 128, 2, 128), lambda i: (i, 0, 0)),
                pl.BlockSpec((W,), lambda i: (i,)),
            ],
            core_axis_name=("c", "s"),
            dimension_semantics=(pltpu.PARALLEL,),
        )(s_hbm, pos_hbm, idx_hbm)

    pos_tiles, idx, _ = run(s, table_t)
    pos = pos_tiles.swapaxes(1, 2).reshape(b, 2)
    return pos, idx


# speedup vs baseline: 1.0035x; 1.0035x over previous
import dataclasses
import functools
import jax
import jax.numpy as jnp
from jax import lax
from jax.experimental import pallas as pl
from jax.experimental.pallas import tpu as pltpu
from jax.experimental.pallas import tpu_sc as plsc

ROW = 8
LANES = 16
W = 2048
SLAB = 256
NSLAB = W // SLAB
LOOK = 6
TCHUNK = 2096


def kernel(s, arclengths, curve_control_points):
    n_seg = curve_control_points.shape[0]
    b = s.shape[0]
    table_t = curve_control_points.reshape(n_seg, ROW).T
    mesh = plsc.VectorSubcoreMesh(core_axis_name="c", subcore_axis_name="s")
    cp = pltpu.CompilerParams()
    if "needs_layout_passes" in pltpu.CompilerParams.__dataclass_fields__:
        cp = dataclasses.replace(cp, needs_layout_passes=False)
    if "use_tc_tiling_on_sc" in pltpu.CompilerParams.__dataclass_fields__:
        cp = dataclasses.replace(cp, use_tc_tiling_on_sc=False)

    @functools.partial(
        pl.kernel,
        mesh=mesh,
        compiler_params=cp,
        out_type=(
            jax.ShapeDtypeStruct((b // 128, 2, 128), jnp.float32),
            jax.ShapeDtypeStruct((b,), jnp.int32),
            jax.ShapeDtypeStruct((n_seg, ROW), jnp.float32),
        ),
        scratch_types=[
            pltpu.VMEM((W,), jnp.int32),
            pltpu.VMEM((W, ROW), jnp.float32),
            pltpu.VMEM((2, ROW, TCHUNK), jnp.float32),
            pltpu.VMEM((2, TCHUNK, ROW), jnp.float32),
        ]
        + [pltpu.SemaphoreType.DMA] * (LOOK + 4),
    )
    def run(s_hbm, tt_hbm, pos_hbm, idx_hbm, tbl_hbm, idxs_v, rows_v, tin_v, tout_v, *sems):
        sid = lax.axis_index("s")
        in_sems = sems[LOOK : LOOK + 2]
        out_sems = sems[LOOK + 2 : LOOK + 4]
        trounds = -(-n_seg // (16 * TCHUNK))

        def chunk_start(r):
            cid = r * 16 + sid
            c0 = jnp.minimum(cid * TCHUNK, n_seg - TCHUNK)
            return pl.multiple_of(c0, 8)

        def fetch(r):
            return pltpu.async_copy(
                tt_hbm.at[:, pl.ds(chunk_start(r), TCHUNK)],
                tin_v.at[r % 2],
                in_sems[r % 2],
            )

        h_in = [None] * trounds
        h_out = [None] * trounds
        h_in[0] = fetch(0)
        for r in range(trounds):
            if r + 1 < trounds:
                h_in[r + 1] = fetch(r + 1)
            h_in[r].wait()
            if r >= 2:
                h_out[r - 2].wait()

            @pl.loop(0, TCHUNK, step=LANES)
            def _(o):
                rid = o + lax.iota(jnp.int32, LANES)
                for j in range(ROW):
                    v = tin_v[r % 2, j, pl.ds(o, LANES)]
                    plsc.store_scatter(
                        tout_v.at[r % 2], [rid, jnp.full((LANES,), j, jnp.int32)], v
                    )

            h_out[r] = pltpu.async_copy(
                tout_v.at[r % 2],
                tbl_hbm.at[pl.ds(chunk_start(r), TCHUNK)],
                out_sems[r % 2],
            )
        for r in range(max(0, trounds - 2), trounds):
            h_out[r].wait()
        plsc.subcore_barrier()

        def body(s_blk, pos_blk, idx_blk):
            def pass_a(k):
                @plsc.parallel_loop(k * SLAB, (k + 1) * SLAB, step=LANES, unroll=2)
                def _(o):
                    sv = s_blk[pl.ds(o, LANES)]
                    ii = jnp.minimum(sv.astype(jnp.int32), n_seg - 1)
                    ii = jnp.maximum(ii, 0)
                    idxs_v[pl.ds(o, LANES)] = ii
                    idx_blk[pl.ds(o, LANES)] = ii

            def issue(k):
                return pltpu.async_copy(
                    tbl_hbm.at[idxs_v.at[pl.ds(k * SLAB, SLAB)]],
                    rows_v.at[pl.ds(k * SLAB, SLAB)],
                    sems[k % LOOK],
                )

            def pass_b(k):
                @plsc.parallel_loop(k * SLAB, (k + 1) * SLAB, step=LANES, unroll=2)
                def _(o):
                    sv = s_blk[pl.ds(o, LANES)]
                    fi = idxs_v[pl.ds(o, LANES)].astype(jnp.float32)
                    t = sv - fi
                    u = 1.0 - t
                    t2 = t * t
                    u2 = u * u
                    b0 = u2 * u
                    b1 = 3.0 * t * u2
                    b2 = 3.0 * t2 * u
                    b3 = t2 * t
                    rid = o + lax.iota(jnp.int32, LANES)
                    c = [
                        plsc.load_gather(
                            rows_v, [rid, jnp.full((LANES,), j, jnp.int32)]
                        )
                        for j in range(ROW)
                    ]
                    px = b0 * c[0] + b1 * c[2] + b2 * c[4] + b3 * c[6]
                    py = b0 * c[1] + b1 * c[3] + b2 * c[5] + b3 * c[7]
                    blk = o // 128
                    off = o - blk * 128
                    pos_blk[blk, 0, pl.ds(off, LANES)] = px
                    pos_blk[blk, 1, pl.ds(off, LANES)] = py

            handles = [None] * NSLAB
            for k in range(NSLAB):
                pass_a(k)
                if k >= LOOK:
                    handles[k - LOOK].wait()
                handles[k] = issue(k)
                if k >= LOOK:
                    pass_b(k - LOOK)
            for k in range(NSLAB - LOOK, NSLAB):
                handles[k].wait()
                pass_b(k)

        pltpu.emit_pipeline(
            body,
            grid=(b // W,),
            in_specs=[pl.BlockSpec((W,), lambda i: (i,))],
            out_specs=[
                pl.BlockSpec((W // 128, 2, 128), lambda i: (i, 0, 0)),
                pl.BlockSpec((W,), lambda i: (i,)),
            ],
            core_axis_name=("c", "s"),
            dimension_semantics=(pltpu.PARALLEL,),
        )(s_hbm, pos_hbm, idx_hbm)

    pos_tiles, idx, _ = run(s, table_t)
    pos = pos_tiles.swapaxes(1, 2).reshape(b, 2)
    return pos, idx
